# Initial kernel scaffold; baseline (speedup 1.0000x reference)
import jax
import jax.numpy as jnp
from jax.experimental import pallas as pl


def kernel(vertex_features, edge_index, W1, b1, g1, be1, W2, b2, g2, be2):
    raise NotImplementedError("write your pallas kernel here")


# speedup vs baseline: 1.8176x; 1.8176x over previous
import functools

import jax
import jax.numpy as jnp
from jax import lax
from jax.experimental import pallas as pl
from jax.experimental.pallas import tpu as pltpu
from jax.experimental.pallas import tpu_sc as plsc

NC = 2
NS = 16
NW = NC * NS
L = 16
BG = 128
NPT = 320
PK = 19
FL = 64
SS = 80


def _node_proj_body(x_ref, w1_ref, p_ref, q_ref):
    x = x_ref[...]
    w1 = w1_ref[...]
    d = x.shape[1]
    wb = w1[d:, :]
    wa = w1[:d, :] - wb
    p_ref[...] = jnp.dot(x, wa, preferred_element_type=jnp.float32)
    q_ref[...] = jnp.dot(x, wb, preferred_element_type=jnp.float32)


def _ln(h, g, b, eps=1e-5):
    mu = jnp.mean(h, axis=-1, keepdims=True)
    hc = h - mu
    var = jnp.mean(hc * hc, axis=-1, keepdims=True)
    return hc * lax.rsqrt(var + eps) * g + b


def _mlp_body(g_ref, w2_ref, b1_ref, g1_ref, be1_ref, b2_ref, g2_ref, be2_ref,
              h_ref):
    h = g_ref[...] + b1_ref[...]
    h = jnp.maximum(_ln(h, g1_ref[...], be1_ref[...]), 0.0)
    h = jnp.dot(h, w2_ref[...], preferred_element_type=jnp.float32) + b2_ref[...]
    h_ref[...] = jnp.maximum(_ln(h, g2_ref[...], be2_ref[...]), 0.0)


def _make_gather_route(nb, epad, d):
    mesh = plsc.VectorSubcoreMesh(core_axis_name="c", subcore_axis_name="s")
    cap = nb * BG
    stride = cap + L

    def body(dst_hbm, src_hbm, p_hbm, q_hbm, g_hbm, list_hbm, cnt_hbm,
             dbuf, sbuf, rowd, rows, stag, cstag, cur_s, nf_s, semd, sems):
        wid = lax.axis_index("s") * NC + lax.axis_index("c")
        pltpu.sync_copy(dst_hbm.at[wid], dbuf)
        pltpu.sync_copy(src_hbm.at[wid], sbuf)
        base = wid * cap

        def zc(o, c):
            cur_s[o] = 0
            nf_s[o] = 0
            return c

        lax.fori_loop(0, NW, zc, 0)

        def batch(j, c0):
            cpd = pltpu.async_copy(p_hbm.at[dbuf.at[j]], rowd, semd)
            cps = pltpu.async_copy(q_hbm.at[sbuf.at[j]], rows, sems)

            for r16 in range(BG // L):
                dvec = dbuf[j, pl.ds(r16 * L, L)]
                for r in range(L):
                    dv = dvec[r]
                    o = (dv * 6554) >> 21
                    pk = ((dv - o * NPT) << PK) | (base + j * BG + r16 * L + r)
                    cur = cur_s[o]
                    stag[pl.ds(o * SS + cur, L)] = jnp.full((L,), pk, jnp.int32)
                    ncur = cur + 1

                    @pl.when(ncur == FL)
                    def _flush():
                        nf = nf_s[o]
                        pltpu.sync_copy(
                            stag.at[pl.ds(o * SS, FL)],
                            list_hbm.at[pl.ds((wid * NW + o) * stride + nf * FL,
                                              FL)])
                        nf_s[o] = nf + 1

                    cur_s[o] = jnp.where(ncur == FL, 0, ncur)

            cpd.wait()
            cps.wait()

            def row(r, c2):
                for jj in range(d // L):
                    sl = pl.ds(jj * L, L)
                    rowd[r, sl] = rowd[r, sl] + rows[r, sl]
                return c2

            lax.fori_loop(0, BG, row, 0)
            pltpu.sync_copy(rowd, g_hbm.at[pl.ds(base + j * BG, BG)])
            return c0

        lax.fori_loop(0, nb, batch, 0)

        for o in range(NW):
            cur = cur_s[o]
            nf = nf_s[o]

            @pl.when(cur > 0)
            def _ff():
                nfl = (cur + L - 1) // L

                def f2(i, c):
                    pltpu.sync_copy(
                        stag.at[pl.ds(o * SS + i * L, L)],
                        list_hbm.at[pl.ds((wid * NW + o) * stride + nf * FL
                                          + i * L, L)])
                    return c

                lax.fori_loop(0, nfl, f2, 0)

            cstag[pl.ds(o, L)] = jnp.full((L,), nf * FL + cur, jnp.int32)
        pltpu.sync_copy(cstag.at[pl.ds(0, NW)], cnt_hbm.at[pl.ds(wid * NW, NW)])

    return pl.kernel(
        body,
        out_type=[jax.ShapeDtypeStruct((epad, d), jnp.float32),
                  jax.ShapeDtypeStruct((NW * NW * stride,), jnp.int32),
                  jax.ShapeDtypeStruct((NW * NW,), jnp.int32)],
        mesh=mesh,
        scratch_types=[
            pltpu.VMEM((nb, BG), jnp.int32),
            pltpu.VMEM((nb, BG), jnp.int32),
            pltpu.VMEM((BG, d), jnp.float32),
            pltpu.VMEM((BG, d), jnp.float32),
            pltpu.VMEM((NW * SS,), jnp.int32),
            pltpu.VMEM((NW + L,), jnp.int32),
            pltpu.SMEM((NW,), jnp.int32),
            pltpu.SMEM((NW,), jnp.int32),
            pltpu.SemaphoreType.DMA,
            pltpu.SemaphoreType.DMA,
        ],
    )


def _make_segmax(nb, d):
    mesh = plsc.VectorSubcoreMesh(core_axis_name="c", subcore_axis_name="s")
    cap = nb * BG
    stride = cap + L

    def body(h_hbm, list_hbm, cnt_hbm, out_hbm, acc, listbuf, cntbuf, idxref,
             rows16, sem):
        wid = lax.axis_index("s") * NC + lax.axis_index("c")
        zf = jnp.zeros((L,), jnp.float32)

        def init_row(r, carry):
            for jj in range(d // L):
                acc[r, pl.ds(jj * L, L)] = zf
            return carry

        lax.fori_loop(0, NPT, init_row, 0)
        pltpu.sync_copy(cnt_hbm, cntbuf.at[pl.ds(0, NW * NW)])

        def per_src(t, carry):
            rid = t * NW + wid
            c = cntbuf[pl.ds(rid, L)][0]
            pltpu.sync_copy(list_hbm.at[pl.ds(rid * stride, stride)], listbuf)
            nbatch = (c + L - 1) // L

            def batch(i, c2):
                pv = listbuf[pl.ds(i * L, L)]
                idxref[pl.ds(0, L)] = pv & ((1 << PK) - 1)
                pltpu.async_copy(h_hbm.at[idxref], rows16, sem).wait()
                for r in range(L):
                    dloc = pv[r] >> PK
                    for jj in range(d // L):
                        sl = pl.ds(jj * L, L)
                        acc[dloc, sl] = jnp.maximum(acc[dloc, sl],
                                                    rows16[r, sl])
                return c2

            lax.fori_loop(0, nbatch, batch, 0)
            return carry

        lax.fori_loop(0, NW, per_src, 0)
        pltpu.sync_copy(acc.at[pl.ds(0, NPT)], out_hbm.at[pl.ds(wid * NPT, NPT)])

    return pl.kernel(
        body,
        out_type=jax.ShapeDtypeStruct((NW * NPT, d), jnp.float32),
        mesh=mesh,
        scratch_types=[
            pltpu.VMEM((NPT, d), jnp.float32),
            pltpu.VMEM((stride,), jnp.int32),
            pltpu.VMEM((NW * NW + L,), jnp.int32),
            pltpu.VMEM((L,), jnp.int32),
            pltpu.VMEM((L, d), jnp.float32),
            pltpu.SemaphoreType.DMA,
        ],
    )


def kernel(vertex_features, edge_index, W1, b1, g1, be1, W2, b2, g2, be2):
    n, d = vertex_features.shape
    e = edge_index.shape[1]
    nb = -(-e // (NW * BG))
    epad = NW * BG * nb
    npad = NW * NPT

    src = edge_index[0]
    dst = edge_index[1]
    pad = epad - e
    if pad:
        src = jnp.concatenate([src, jnp.zeros((pad,), jnp.int32)])
        dst = jnp.concatenate([dst, jnp.full((pad,), n, jnp.int32)])
    x_pad = jnp.concatenate(
        [vertex_features, jnp.zeros((npad - n, d), jnp.float32)])

    p_tab, q_tab = pl.pallas_call(
        _node_proj_body,
        out_shape=[jax.ShapeDtypeStruct((npad, d), jnp.float32),
                   jax.ShapeDtypeStruct((npad, d), jnp.float32)],
    )(x_pad, W1)

    g_edges, lists, counts = _make_gather_route(nb, epad, d)(
        dst.reshape(NW, nb, BG), src.reshape(NW, nb, BG), p_tab, q_tab)

    bt = 1024
    nblk = epad // bt
    vspec = pl.BlockSpec((1, d), lambda i: (0, 0))
    h_edges = pl.pallas_call(
        _mlp_body,
        grid=(nblk,),
        in_specs=[pl.BlockSpec((bt, d), lambda i: (i, 0)),
                  pl.BlockSpec((d, d), lambda i: (0, 0)),
                  vspec, vspec, vspec, vspec, vspec, vspec],
        out_specs=pl.BlockSpec((bt, d), lambda i: (i, 0)),
        out_shape=jax.ShapeDtypeStruct((epad, d), jnp.float32),
    )(g_edges, W2, b1.reshape(1, d), g1.reshape(1, d), be1.reshape(1, d),
      b2.reshape(1, d), g2.reshape(1, d), be2.reshape(1, d))

    out_pad = _make_segmax(nb, d)(h_edges, lists, counts)
    return out_pad[:n]


# speedup vs baseline: 1.9340x; 1.0641x over previous
import functools

import jax
import jax.numpy as jnp
from jax import lax
from jax.experimental import pallas as pl
from jax.experimental.pallas import tpu as pltpu
from jax.experimental.pallas import tpu_sc as plsc

NC = 2
NS = 16
NW = NC * NS
L = 16
BG = 128
NPT = 320
PK = 19
FL = 64
SS = 80


def _node_proj_body(x_ref, w1_ref, p_ref, q_ref):
    x = x_ref[...]
    w1 = w1_ref[...]
    d = x.shape[1]
    wb = w1[d:, :]
    wa = w1[:d, :] - wb
    p_ref[...] = jnp.dot(x, wa, preferred_element_type=jnp.float32)
    q_ref[...] = jnp.dot(x, wb, preferred_element_type=jnp.float32)


def _ln(h, g, b, eps=1e-5):
    mu = jnp.mean(h, axis=-1, keepdims=True)
    hc = h - mu
    var = jnp.mean(hc * hc, axis=-1, keepdims=True)
    return hc * lax.rsqrt(var + eps) * g + b


def _mlp_body(gd_ref, gs_ref, w2_ref, b1_ref, g1_ref, be1_ref, b2_ref, g2_ref,
              be2_ref, h_ref):
    h = gd_ref[...] + gs_ref[...] + b1_ref[...]
    h = jnp.maximum(_ln(h, g1_ref[...], be1_ref[...]), 0.0)
    h = jnp.dot(h, w2_ref[...], preferred_element_type=jnp.float32) + b2_ref[...]
    h_ref[...] = jnp.maximum(_ln(h, g2_ref[...], be2_ref[...]), 0.0)


def _make_gather_route(nb, epad, d):
    mesh = plsc.VectorSubcoreMesh(core_axis_name="c", subcore_axis_name="s")
    cap = nb * BG
    stride = cap + L

    def body(dst_hbm, src_hbm, p_hbm, q_hbm, gd_hbm, gs_hbm, list_hbm, cnt_hbm,
             dbuf, sbuf, pda, psa, pdb, psb, stag, cstag, cur_s, nf_s,
             semda, semsa, semdb, semsb):
        wid = lax.axis_index("s") * NC + lax.axis_index("c")
        pltpu.sync_copy(dst_hbm.at[wid], dbuf)
        pltpu.sync_copy(src_hbm.at[wid], sbuf)
        base = wid * cap

        def zc(o, c):
            cur_s[o] = 0
            nf_s[o] = 0
            return c

        lax.fori_loop(0, NW, zc, 0)

        def route(j):
            for r16 in range(BG // L):
                dvec = dbuf[j, pl.ds(r16 * L, L)]
                for r in range(L):
                    dv = dvec[r]
                    o = (dv * 6554) >> 21
                    pk = ((dv - o * NPT) << PK) | (base + j * BG + r16 * L + r)
                    cur = cur_s[o]
                    stag[pl.ds(o * SS + cur, L)] = jnp.full((L,), pk, jnp.int32)
                    ncur = cur + 1

                    @pl.when(ncur == FL)
                    def _flush():
                        nf = nf_s[o]
                        pltpu.sync_copy(
                            stag.at[pl.ds(o * SS, FL)],
                            list_hbm.at[pl.ds((wid * NW + o) * stride + nf * FL,
                                              FL)])
                        nf_s[o] = nf + 1

                    cur_s[o] = jnp.where(ncur == FL, 0, ncur)

        def fire(j, pd, ps, semd, sems):
            pltpu.async_copy(p_hbm.at[dbuf.at[j]], pd, semd)
            pltpu.async_copy(q_hbm.at[sbuf.at[j]], ps, sems)

        def wait(j, pd, ps, semd, sems):
            pltpu.make_async_copy(p_hbm.at[dbuf.at[j]], pd, semd).wait()
            pltpu.make_async_copy(q_hbm.at[sbuf.at[j]], ps, sems).wait()

        def write(j, pd, ps):
            pltpu.sync_copy(pd, gd_hbm.at[pl.ds(base + j * BG, BG)])
            pltpu.sync_copy(ps, gs_hbm.at[pl.ds(base + j * BG, BG)])

        fire(0, pda, psa, semda, semsa)

        def pair(jp, c0):
            j0 = 2 * jp
            j1 = j0 + 1
            fire(j1, pdb, psb, semdb, semsb)
            route(j0)
            wait(j0, pda, psa, semda, semsa)
            write(j0, pda, psa)
            fire(jnp.minimum(j0 + 2, nb - 1), pda, psa, semda, semsa)
            route(j1)
            wait(j1, pdb, psb, semdb, semsb)
            write(j1, pdb, psb)
            return c0

        lax.fori_loop(0, nb // 2, pair, 0)
        wait(nb - 1, pda, psa, semda, semsa)

        for o in range(NW):
            cur = cur_s[o]
            nf = nf_s[o]

            @pl.when(cur > 0)
            def _ff():
                nfl = (cur + L - 1) // L

                def f2(i, c):
                    pltpu.sync_copy(
                        stag.at[pl.ds(o * SS + i * L, L)],
                        list_hbm.at[pl.ds((wid * NW + o) * stride + nf * FL
                                          + i * L, L)])
                    return c

                lax.fori_loop(0, nfl, f2, 0)

            cstag[pl.ds(o, L)] = jnp.full((L,), nf * FL + cur, jnp.int32)
        pltpu.sync_copy(cstag.at[pl.ds(0, NW)], cnt_hbm.at[pl.ds(wid * NW, NW)])

    return pl.kernel(
        body,
        out_type=[jax.ShapeDtypeStruct((epad, d), jnp.float32),
                  jax.ShapeDtypeStruct((epad, d), jnp.float32),
                  jax.ShapeDtypeStruct((NW * NW * stride,), jnp.int32),
                  jax.ShapeDtypeStruct((NW * NW,), jnp.int32)],
        mesh=mesh,
        scratch_types=[
            pltpu.VMEM((nb, BG), jnp.int32),
            pltpu.VMEM((nb, BG), jnp.int32),
            pltpu.VMEM((BG, d), jnp.float32),
            pltpu.VMEM((BG, d), jnp.float32),
            pltpu.VMEM((BG, d), jnp.float32),
            pltpu.VMEM((BG, d), jnp.float32),
            pltpu.VMEM((NW * SS,), jnp.int32),
            pltpu.VMEM((NW + L,), jnp.int32),
            pltpu.SMEM((NW,), jnp.int32),
            pltpu.SMEM((NW,), jnp.int32),
            pltpu.SemaphoreType.DMA,
            pltpu.SemaphoreType.DMA,
            pltpu.SemaphoreType.DMA,
            pltpu.SemaphoreType.DMA,
        ],
    )


def _make_segmax(nb, d):
    mesh = plsc.VectorSubcoreMesh(core_axis_name="c", subcore_axis_name="s")
    cap = nb * BG
    stride = cap + L
    epad = NW * cap

    def body(h_hbm, list_hbm, cnt_hbm, out_hbm, acc, listbuf, cntbuf, idxa,
             idxb, rowsa, rowsb, sema, semb):
        wid = lax.axis_index("s") * NC + lax.axis_index("c")
        zf = jnp.zeros((L,), jnp.float32)
        emask = (1 << PK) - 1

        def init_row(r, carry):
            for jj in range(d // L):
                acc[r, pl.ds(jj * L, L)] = zf
            return carry

        lax.fori_loop(0, NPT, init_row, 0)
        pltpu.sync_copy(cnt_hbm, cntbuf.at[pl.ds(0, NW * NW)])

        def fire(i, idxr, rows, sem):
            pv = listbuf[pl.ds(i * L, L)]
            idxr[pl.ds(0, L)] = jnp.minimum(pv & emask, epad - 1)
            pltpu.async_copy(h_hbm.at[idxr], rows, sem)

        def wait(idxr, rows, sem):
            pltpu.make_async_copy(h_hbm.at[idxr], rows, sem).wait()

        def update(i, rows):
            pv = listbuf[pl.ds(i * L, L)]
            for r in range(L):
                dloc = pv[r] >> PK
                for jj in range(d // L):
                    sl = pl.ds(jj * L, L)
                    acc[dloc, sl] = jnp.maximum(acc[dloc, sl], rows[r, sl])

        def per_src(t, carry):
            rid = t * NW + wid
            c = cntbuf[pl.ds(rid, L)][0]

            @pl.when(c > 0)
            def _go():
                pltpu.sync_copy(list_hbm.at[pl.ds(rid * stride, stride)],
                                listbuf)
                nbatch = (c + L - 1) // L
                fire(0, idxa, rowsa, sema)

                def bpair(p, c2):
                    i0 = 2 * p
                    i1 = i0 + 1
                    fire(jnp.minimum(i1, nbatch - 1), idxb, rowsb, semb)
                    wait(idxa, rowsa, sema)
                    update(i0, rowsa)
                    fire(jnp.minimum(i0 + 2, nbatch - 1), idxa, rowsa, sema)
                    wait(idxb, rowsb, semb)

                    @pl.when(i1 < nbatch)
                    def _u1():
                        update(i1, rowsb)

                    return c2

                lax.fori_loop(0, (nbatch + 1) // 2, bpair, 0)
                wait(idxa, rowsa, sema)

            return carry

        lax.fori_loop(0, NW, per_src, 0)
        pltpu.sync_copy(acc.at[pl.ds(0, NPT)], out_hbm.at[pl.ds(wid * NPT, NPT)])

    return pl.kernel(
        body,
        out_type=jax.ShapeDtypeStruct((NW * NPT, d), jnp.float32),
        mesh=mesh,
        scratch_types=[
            pltpu.VMEM((NPT, d), jnp.float32),
            pltpu.VMEM((stride,), jnp.int32),
            pltpu.VMEM((NW * NW + L,), jnp.int32),
            pltpu.VMEM((L,), jnp.int32),
            pltpu.VMEM((L,), jnp.int32),
            pltpu.VMEM((L, d), jnp.float32),
            pltpu.VMEM((L, d), jnp.float32),
            pltpu.SemaphoreType.DMA,
            pltpu.SemaphoreType.DMA,
        ],
    )


def kernel(vertex_features, edge_index, W1, b1, g1, be1, W2, b2, g2, be2):
    n, d = vertex_features.shape
    e = edge_index.shape[1]
    nb = -(-e // (NW * BG))
    nb += nb % 2
    epad = NW * BG * nb
    npad = NW * NPT

    src = edge_index[0]
    dst = edge_index[1]
    pad = epad - e
    if pad:
        src = jnp.concatenate([src, jnp.zeros((pad,), jnp.int32)])
        dst = jnp.concatenate([dst, jnp.full((pad,), n, jnp.int32)])
    x_pad = jnp.concatenate(
        [vertex_features, jnp.zeros((npad - n, d), jnp.float32)])

    p_tab, q_tab = pl.pallas_call(
        _node_proj_body,
        out_shape=[jax.ShapeDtypeStruct((npad, d), jnp.float32),
                   jax.ShapeDtypeStruct((npad, d), jnp.float32)],
    )(x_pad, W1)

    gd_edges, gs_edges, lists, counts = _make_gather_route(nb, epad, d)(
        dst.reshape(NW, nb, BG), src.reshape(NW, nb, BG), p_tab, q_tab)

    bt = 1024
    nblk = epad // bt
    vspec = pl.BlockSpec((1, d), lambda i: (0, 0))
    espec = pl.BlockSpec((bt, d), lambda i: (i, 0))
    h_edges = pl.pallas_call(
        _mlp_body,
        grid=(nblk,),
        in_specs=[espec, espec,
                  pl.BlockSpec((d, d), lambda i: (0, 0)),
                  vspec, vspec, vspec, vspec, vspec, vspec],
        out_specs=espec,
        out_shape=jax.ShapeDtypeStruct((epad, d), jnp.float32),
    )(gd_edges, gs_edges, W2, b1.reshape(1, d), g1.reshape(1, d),
      be1.reshape(1, d), b2.reshape(1, d), g2.reshape(1, d), be2.reshape(1, d))

    out_pad = _make_segmax(nb, d)(h_edges, lists, counts)
    return out_pad[:n]
